# baseline (speedup 1.0000x reference)
import functools

import jax
import jax.numpy as jnp
from jax import lax
from jax.experimental import pallas as pl
from jax.experimental.pallas import tpu as pltpu
from jax.experimental.pallas import tpu_sc as plsc

N = 10000
E = 160000
D = 256
HID = 256
C = 64
LAT = 128
OUT = 1

BLK = 2000
NW = 32
NCH = 40
CH = 128
E_PAD = NW * NCH * CH
ROWS_PER_SUB = N // 16


def _tc1_body(x_ref, w1_ref, b1_ref, w2_ref, b2_ref, aw1_ref, ab1_ref,
              aw2_ref, ab2_ref, a_ref, abf_ref, cn_ref):
    x = x_ref[...]
    h = jnp.maximum(jnp.dot(x, w1_ref[...], preferred_element_type=jnp.float32)
                    + b1_ref[...], 0.0)
    h = jnp.dot(h, w2_ref[...], preferred_element_type=jnp.float32) + b2_ref[...]
    l1 = jnp.maximum(jnp.dot(h, aw1_ref[...], preferred_element_type=jnp.float32)
                     + ab1_ref[...], 0.0)
    logits = (jnp.dot(l1, aw2_ref[...], preferred_element_type=jnp.float32)
              + ab2_ref[...])
    m = jnp.max(logits, axis=-1, keepdims=True)
    ex = jnp.exp(logits - m)
    a = ex / jnp.sum(ex, axis=-1, keepdims=True)
    a_ref[...] = a
    abf_ref[...] = a.astype(jnp.bfloat16)
    part = lax.dot_general(a, h, (((0,), (0,)), ((), ())),
                           preferred_element_type=jnp.float32)

    @pl.when(pl.program_id(0) == 0)
    def _():
        cn_ref[...] = part

    @pl.when(pl.program_id(0) != 0)
    def _():
        cn_ref[...] += part


def _tc1(nodes, enc_W1, enc_b1, enc_W2, enc_b2, asn_W1, asn_b1, asn_W2, asn_b2):
    full = lambda shape: pl.BlockSpec(shape, lambda i: (0,) * len(shape))
    return pl.pallas_call(
        _tc1_body,
        grid=(N // BLK,),
        in_specs=[
            pl.BlockSpec((BLK, D), lambda i: (i, 0)),
            full((D, HID)), full((1, HID)), full((HID, HID)), full((1, HID)),
            full((HID, 32)), full((1, 32)), full((32, C)), full((1, C)),
        ],
        out_specs=[
            pl.BlockSpec((BLK, C), lambda i: (i, 0)),
            pl.BlockSpec((BLK, C), lambda i: (i, 0)),
            full((C, HID)),
        ],
        out_shape=[
            jax.ShapeDtypeStruct((N, C), jnp.float32),
            jax.ShapeDtypeStruct((N, C), jnp.bfloat16),
            jax.ShapeDtypeStruct((C, HID), jnp.float32),
        ],
    )(nodes, enc_W1, enc_b1, enc_W2, enc_b2, asn_W1, asn_b1, asn_W2, asn_b2)


NBUF = 4


def _sc1_body(send_hbm, recv_hbm, ew_hbm, abf_hbm, t_out,
              s_v, r_v, w_v, gbufs, sbufs, a_sh, t_sh, gsems, ssems):
    cid = lax.axis_index("c")
    sid = lax.axis_index("s")
    w = cid * 16 + sid

    def _zrow(i, _):
        def _zcol(q, _):
            sbufs[0][i, pl.ds(q * 16, 16)] = jnp.zeros((16,), jnp.float32)
            return 0
        return lax.fori_loop(0, C // 16, _zcol, 0)
    lax.fori_loop(0, CH, _zrow, 0)
    for k in range(ROWS_PER_SUB // 125):
        pltpu.sync_copy(sbufs[0].at[pl.ds(0, 125)],
                        t_sh.at[pl.ds(sid * ROWS_PER_SUB + k * 125, 125)])
    pltpu.sync_copy(abf_hbm.at[pl.ds(sid * ROWS_PER_SUB, ROWS_PER_SUB)],
                    a_sh.at[pl.ds(sid * ROWS_PER_SUB, ROWS_PER_SUB)])
    plsc.subcore_barrier()

    pltpu.sync_copy(send_hbm.at[w], s_v)
    pltpu.sync_copy(recv_hbm.at[w], r_v)
    pltpu.sync_copy(ew_hbm.at[w], w_v)

    def _gather(j, b):
        return pltpu.make_async_copy(a_sh.at[r_v.at[j]], gbufs[b], gsems[b])

    def _scatter(j, b):
        return pltpu.make_async_copy(sbufs[b], t_sh.at[s_v.at[j]], ssems[b])

    for b in range(NBUF):
        _gather(b, b).start()

    def _outer(j4, _):
        for b in range(NBUF):
            j = j4 * NBUF + b
            _gather(j, b).wait()

            @pl.when(j >= NBUF)
            def _():
                _scatter(j - NBUF, b).wait()

            def _group(g, _):
                w16 = w_v[j, pl.ds(g * 16, 16)]
                for u in range(16):
                    ws = w16[u]
                    row = g * 16 + u
                    for q in range(C // 32):
                        x = gbufs[b][row, pl.ds(q * 32, 32)]
                        lo, hi = plsc.unpack(
                            x, format=plsc.PackFormat.INTERLEAVED)
                        sbufs[b][row, pl.ds(q * 32, 16)] = lo * ws
                        sbufs[b][row, pl.ds(q * 32 + 16, 16)] = hi * ws
                return 0
            lax.fori_loop(0, CH // 16, _group, 0, unroll=4)

            @pl.when(j + NBUF < NCH)
            def _():
                _gather(j + NBUF, b).start()
            _scatter(j, b).start(add=True)
        return 0
    lax.fori_loop(0, NCH // NBUF, _outer, 0)
    for b in range(NBUF):
        jtail = NCH - 1 - ((NCH - 1 - b) % NBUF)
        _scatter(jtail, b).wait()
    plsc.subcore_barrier()

    pltpu.sync_copy(t_sh.at[pl.ds(sid * ROWS_PER_SUB, ROWS_PER_SUB)],
                    t_out.at[pl.ds(sid * ROWS_PER_SUB, ROWS_PER_SUB),
                             pl.ds(cid * C, C)])


_sc1 = functools.partial(
    pl.kernel,
    out_type=jax.ShapeDtypeStruct((N, 2 * C), jnp.float32),
    mesh=plsc.VectorSubcoreMesh(core_axis_name="c", subcore_axis_name="s"),
    scratch_types=[
        pltpu.VMEM((NCH, CH), jnp.int32),
        pltpu.VMEM((NCH, CH), jnp.int32),
        pltpu.VMEM((NCH, CH), jnp.float32),
        [pltpu.VMEM((CH, C), jnp.bfloat16)] * NBUF,
        [pltpu.VMEM((CH, C), jnp.float32)] * NBUF,
        pltpu.VMEM_SHARED((N, C), jnp.bfloat16),
        pltpu.VMEM_SHARED((N, C), jnp.float32),
        [pltpu.SemaphoreType.DMA] * NBUF,
        [pltpu.SemaphoreType.DMA] * NBUF,
    ],
    compiler_params=pltpu.CompilerParams(needs_layout_passes=False, use_tc_tiling_on_sc=False),
)(_sc1_body)


def _tc2_body(a_ref, t_ref, cn_ref, muW_ref, mub_ref, lvW_ref, lvb_ref,
              pw1_ref, pb1_ref, pw2_ref, pb2_ref, eps_ref,
              adj_ref, mu_ref, lv_ref, py_ref, nnz_ref):
    i = pl.program_id(0)
    t = t_ref[:, :C] + t_ref[:, C:]
    part = lax.dot_general(a_ref[...], t, (((0,), (0,)), ((), ())),
                           preferred_element_type=jnp.float32)

    @pl.when(i == 0)
    def _():
        adj_ref[...] = part

    @pl.when(i != 0)
    def _():
        adj_ref[...] += part

    @pl.when(i == pl.num_programs(0) - 1)
    def _():
        rowi = lax.broadcasted_iota(jnp.int32, (C, C), 0)
        colj = lax.broadcasted_iota(jnp.int32, (C, C), 1)
        cvals = ((rowi // 32) * 32 + 2 * (rowi % 16)
                 + jnp.where((rowi % 32) >= 16, 1, 0))
        perm = (colj == cvals).astype(jnp.float32)
        adjf = jnp.dot(adj_ref[...], perm,
                       preferred_element_type=jnp.float32)
        adj_ref[...] = adjf
        macro = jnp.mean(cn_ref[...], axis=0, keepdims=True)
        mu = jnp.dot(macro, muW_ref[...],
                     preferred_element_type=jnp.float32) + mub_ref[...]
        lv = jnp.dot(macro, lvW_ref[...],
                     preferred_element_type=jnp.float32) + lvb_ref[...]
        std = jnp.exp(0.5 * lv)
        z = mu + eps_ref[...] * std
        p1 = jnp.maximum(jnp.dot(z, pw1_ref[...],
                                 preferred_element_type=jnp.float32)
                         + pb1_ref[...], 0.0)
        py = jnp.dot(p1, pw2_ref[...],
                     preferred_element_type=jnp.float32) + pb2_ref[...]
        mu_ref[...] = mu
        lv_ref[...] = lv
        py_ref[...] = py
        nnz_ref[...] = jnp.sum(
            (adjf != 0.0).astype(jnp.float32)).reshape(1, 1)


def _tc2(a, t, cn, mu_W, mu_b, lv_W, lv_b, pr_W1, pr_b1, pr_W2, pr_b2, eps):
    full = lambda shape: pl.BlockSpec(shape, lambda i: (0,) * len(shape))
    return pl.pallas_call(
        _tc2_body,
        grid=(N // BLK,),
        in_specs=[
            pl.BlockSpec((BLK, C), lambda i: (i, 0)),
            pl.BlockSpec((BLK, 2 * C), lambda i: (i, 0)),
            full((C, HID)),
            full((HID, LAT)), full((1, LAT)), full((HID, LAT)), full((1, LAT)),
            full((LAT, 32)), full((1, 32)), full((32, OUT)), full((1, OUT)),
            full((1, LAT)),
        ],
        out_specs=[full((C, C)), full((1, LAT)), full((1, LAT)),
                   full((1, OUT)), full((1, 1))],
        out_shape=[
            jax.ShapeDtypeStruct((C, C), jnp.float32),
            jax.ShapeDtypeStruct((1, LAT), jnp.float32),
            jax.ShapeDtypeStruct((1, LAT), jnp.float32),
            jax.ShapeDtypeStruct((1, OUT), jnp.float32),
            jax.ShapeDtypeStruct((1, 1), jnp.float32),
        ],
    )(a, t, cn, mu_W, mu_b, lv_W, lv_b, pr_W1, pr_b1, pr_W2, pr_b2, eps)


def _sc2_body(adj_hbm, out_hbm, v_v, o_v):
    cid = lax.axis_index("c")
    sid = lax.axis_index("s")

    @pl.when(jnp.logical_and(cid == 0, sid == 0))
    def _():
        pltpu.sync_copy(adj_hbm, v_v)
        v0 = v_v[pl.ds(0, 16)]
        fill = jnp.full((16,), v0[0], jnp.float32)

        def _prefill(k, _):
            o_v[pl.ds(k * 16, 16)] = fill
            return 0
        lax.fori_loop(0, (C * C) // 16, _prefill, 0)

        def _pack(k, base):
            x = v_v[pl.ds(k * 16, 16)]
            m = x != 0.0
            pos = base + plsc.cumsum(m.astype(jnp.int32)) - 1
            plsc.store_scatter(o_v, [pos], x, mask=m)
            return base + plsc.all_reduce_population_count(m)
        lax.fori_loop(0, (C * C) // 16, _pack, jnp.zeros((16,), jnp.int32))
        pltpu.sync_copy(o_v, out_hbm)


_sc2 = functools.partial(
    pl.kernel,
    out_type=jax.ShapeDtypeStruct((C * C,), jnp.float32),
    mesh=plsc.VectorSubcoreMesh(core_axis_name="c", subcore_axis_name="s"),
    scratch_types=[
        pltpu.VMEM((C * C,), jnp.float32),
        pltpu.VMEM((C * C,), jnp.float32),
    ],
    compiler_params=pltpu.CompilerParams(needs_layout_passes=False, use_tc_tiling_on_sc=False),
)(_sc2_body)


def kernel(nodes, senders, receivers, edges, enc_W1, enc_b1, enc_W2, enc_b2,
           asn_W1, asn_b1, asn_W2, asn_b2, mu_W, mu_b, lv_W, lv_b,
           pr_W1, pr_b1, pr_W2, pr_b2):
    r2 = lambda b: b.reshape(1, -1)
    assignments, a_bf, coarse_nodes = _tc1(
        nodes, enc_W1, r2(enc_b1), enc_W2, r2(enc_b2),
        asn_W1, r2(asn_b1), asn_W2, r2(asn_b2))

    pad = E_PAD - E
    zi = jnp.zeros((pad,), jnp.int32)
    send3 = jnp.concatenate([senders.astype(jnp.int32), zi]).reshape(NW, NCH, CH)
    recv3 = jnp.concatenate([receivers.astype(jnp.int32), zi]).reshape(NW, NCH, CH)
    ew3 = jnp.concatenate([edges.astype(jnp.float32).reshape(-1),
                           jnp.zeros((pad,), jnp.float32)]).reshape(NW, NCH, CH)
    t_partial = _sc1(send3, recv3, ew3, a_bf)

    eps = jax.random.normal(jax.random.PRNGKey(0), (1, LAT))
    coarse_adj, mu, logvar, pred_y, nnz = _tc2(
        assignments, t_partial, coarse_nodes,
        mu_W, r2(mu_b), lv_W, r2(lv_b),
        pr_W1, r2(pr_b1), pr_W2, r2(pr_b2), eps)

    c_edges = lax.cond(
        nnz[0, 0] == float(C * C),
        lambda a: a.reshape(C * C, 1),
        lambda a: _sc2(a.reshape(C * C)).reshape(C * C, 1),
        coarse_adj)
    return (mu, logvar, pred_y, assignments, coarse_nodes, c_edges)


# speedup vs baseline: 1.5461x; 1.5461x over previous
import functools

import jax
import jax.numpy as jnp
from jax import lax
from jax.experimental import pallas as pl
from jax.experimental.pallas import tpu as pltpu
from jax.experimental.pallas import tpu_sc as plsc

N = 10000
E = 160000
D = 256
HID = 256
C = 64
LAT = 128
OUT = 1

BLK = 2000
NW = 32
NCH = 40
CH = 128
E_PAD = NW * NCH * CH
ROWS_PER_SUB = N // 16


def _tc1_body(x_ref, w1_ref, b1_ref, w2_ref, b2_ref, aw1_ref, ab1_ref,
              aw2_ref, ab2_ref, a_ref, cn_ref):
    x = x_ref[...]
    h = jnp.maximum(jnp.dot(x, w1_ref[...], preferred_element_type=jnp.float32)
                    + b1_ref[...], 0.0)
    h = jnp.dot(h, w2_ref[...], preferred_element_type=jnp.float32) + b2_ref[...]
    l1 = jnp.maximum(jnp.dot(h, aw1_ref[...], preferred_element_type=jnp.float32)
                     + ab1_ref[...], 0.0)
    logits = (jnp.dot(l1, aw2_ref[...], preferred_element_type=jnp.float32)
              + ab2_ref[...])
    m = jnp.max(logits, axis=-1, keepdims=True)
    ex = jnp.exp(logits - m)
    a = ex / jnp.sum(ex, axis=-1, keepdims=True)
    a_ref[...] = a
    part = lax.dot_general(a, h, (((0,), (0,)), ((), ())),
                           preferred_element_type=jnp.float32)

    @pl.when(pl.program_id(0) == 0)
    def _():
        cn_ref[...] = part

    @pl.when(pl.program_id(0) != 0)
    def _():
        cn_ref[...] += part


def _tc1(nodes, enc_W1, enc_b1, enc_W2, enc_b2, asn_W1, asn_b1, asn_W2, asn_b2):
    full = lambda shape: pl.BlockSpec(shape, lambda i: (0,) * len(shape))
    return pl.pallas_call(
        _tc1_body,
        grid=(N // BLK,),
        in_specs=[
            pl.BlockSpec((BLK, D), lambda i: (i, 0)),
            full((D, HID)), full((1, HID)), full((HID, HID)), full((1, HID)),
            full((HID, 32)), full((1, 32)), full((32, C)), full((1, C)),
        ],
        out_specs=[
            pl.BlockSpec((BLK, C), lambda i: (i, 0)),
            full((C, HID)),
        ],
        out_shape=[
            jax.ShapeDtypeStruct((N, C), jnp.float32),
            jax.ShapeDtypeStruct((C, HID), jnp.float32),
        ],
    )(nodes, enc_W1, enc_b1, enc_W2, enc_b2, asn_W1, asn_b1, asn_W2, asn_b2)


NBUF = 4


def _sc1_body(send_hbm, recv_hbm, ew_hbm, a_hbm, t_out,
              s_v, r_v, w_v, gbufs, a_sh, t_sh, gsems, ssems):
    cid = lax.axis_index("c")
    sid = lax.axis_index("s")
    w = cid * 16 + sid

    def _zrow(i, _):
        def _zcol(q, _):
            gbufs[0][i, pl.ds(q * 16, 16)] = jnp.zeros((16,), jnp.float32)
            return 0
        return lax.fori_loop(0, C // 16, _zcol, 0)
    lax.fori_loop(0, CH, _zrow, 0)
    for k in range(ROWS_PER_SUB // 125):
        pltpu.sync_copy(gbufs[0].at[pl.ds(0, 125)],
                        t_sh.at[pl.ds(sid * ROWS_PER_SUB + k * 125, 125)])
    pltpu.sync_copy(a_hbm.at[pl.ds(sid * ROWS_PER_SUB, ROWS_PER_SUB)],
                    a_sh.at[pl.ds(sid * ROWS_PER_SUB, ROWS_PER_SUB)])
    plsc.subcore_barrier()

    pltpu.sync_copy(send_hbm.at[w], s_v)
    pltpu.sync_copy(recv_hbm.at[w], r_v)
    pltpu.sync_copy(ew_hbm.at[w], w_v)

    def _gather(j, b):
        return pltpu.make_async_copy(a_sh.at[r_v.at[j]], gbufs[b], gsems[b])

    def _scatter(j, b):
        return pltpu.make_async_copy(gbufs[b], t_sh.at[s_v.at[j]], ssems[b])

    for b in range(NBUF):
        _gather(b, b).start()

    def _outer(j4, _):
        for b in range(NBUF):
            j = j4 * NBUF + b
            _gather(j, b).wait()

            def _group(g, _):
                w16 = w_v[j, pl.ds(g * 16, 16)]
                for u in range(16):
                    ws = w16[u]
                    row = g * 16 + u
                    for q in range(C // 16):
                        sl = pl.ds(q * 16, 16)
                        gbufs[b][row, sl] = gbufs[b][row, sl] * ws
                return 0
            lax.fori_loop(0, CH // 16, _group, 0, unroll=4)
            _scatter(j, b).start(add=True)

            bp = (b - 1) % NBUF
            jp = j - 1

            @pl.when(jnp.logical_and(jp >= 0, jp + NBUF < NCH))
            def _():
                _scatter(jp, bp).wait()
                _gather(jp + NBUF, bp).start()
        return 0
    lax.fori_loop(0, NCH // NBUF, _outer, 0)
    for b in range(NBUF):
        jtail = NCH - 1 - ((NCH - 1 - b) % NBUF)
        _scatter(jtail, b).wait()
    plsc.subcore_barrier()

    pltpu.sync_copy(t_sh.at[pl.ds(sid * ROWS_PER_SUB, ROWS_PER_SUB)],
                    t_out.at[pl.ds(sid * ROWS_PER_SUB, ROWS_PER_SUB),
                             pl.ds(cid * C, C)])


_sc1 = functools.partial(
    pl.kernel,
    out_type=jax.ShapeDtypeStruct((N, 2 * C), jnp.float32),
    mesh=plsc.VectorSubcoreMesh(core_axis_name="c", subcore_axis_name="s"),
    scratch_types=[
        pltpu.VMEM((NCH, CH), jnp.int32),
        pltpu.VMEM((NCH, CH), jnp.int32),
        pltpu.VMEM((NCH, CH), jnp.float32),
        [pltpu.VMEM((CH, C), jnp.float32)] * NBUF,
        pltpu.VMEM_SHARED((N, C), jnp.float32),
        pltpu.VMEM_SHARED((N, C), jnp.float32),
        [pltpu.SemaphoreType.DMA] * NBUF,
        [pltpu.SemaphoreType.DMA] * NBUF,
    ],
    compiler_params=pltpu.CompilerParams(needs_layout_passes=False, use_tc_tiling_on_sc=False),
)(_sc1_body)


def _tc2_body(a_ref, t_ref, cn_ref, muW_ref, mub_ref, lvW_ref, lvb_ref,
              pw1_ref, pb1_ref, pw2_ref, pb2_ref, eps_ref,
              adj_ref, mu_ref, lv_ref, py_ref, nnz_ref):
    i = pl.program_id(0)
    t = t_ref[:, :C] + t_ref[:, C:]
    part = lax.dot_general(a_ref[...], t, (((0,), (0,)), ((), ())),
                           preferred_element_type=jnp.float32)

    @pl.when(i == 0)
    def _():
        adj_ref[...] = part

    @pl.when(i != 0)
    def _():
        adj_ref[...] += part

    @pl.when(i == pl.num_programs(0) - 1)
    def _():
        adjf = adj_ref[...]
        macro = jnp.mean(cn_ref[...], axis=0, keepdims=True)
        mu = jnp.dot(macro, muW_ref[...],
                     preferred_element_type=jnp.float32) + mub_ref[...]
        lv = jnp.dot(macro, lvW_ref[...],
                     preferred_element_type=jnp.float32) + lvb_ref[...]
        std = jnp.exp(0.5 * lv)
        z = mu + eps_ref[...] * std
        p1 = jnp.maximum(jnp.dot(z, pw1_ref[...],
                                 preferred_element_type=jnp.float32)
                         + pb1_ref[...], 0.0)
        py = jnp.dot(p1, pw2_ref[...],
                     preferred_element_type=jnp.float32) + pb2_ref[...]
        mu_ref[...] = mu
        lv_ref[...] = lv
        py_ref[...] = py
        nnz_ref[...] = jnp.sum(
            (adjf != 0.0).astype(jnp.float32)).reshape(1, 1)


def _tc2(a, t, cn, mu_W, mu_b, lv_W, lv_b, pr_W1, pr_b1, pr_W2, pr_b2, eps):
    full = lambda shape: pl.BlockSpec(shape, lambda i: (0,) * len(shape))
    return pl.pallas_call(
        _tc2_body,
        grid=(N // BLK,),
        in_specs=[
            pl.BlockSpec((BLK, C), lambda i: (i, 0)),
            pl.BlockSpec((BLK, 2 * C), lambda i: (i, 0)),
            full((C, HID)),
            full((HID, LAT)), full((1, LAT)), full((HID, LAT)), full((1, LAT)),
            full((LAT, 32)), full((1, 32)), full((32, OUT)), full((1, OUT)),
            full((1, LAT)),
        ],
        out_specs=[full((C, C)), full((1, LAT)), full((1, LAT)),
                   full((1, OUT)), full((1, 1))],
        out_shape=[
            jax.ShapeDtypeStruct((C, C), jnp.float32),
            jax.ShapeDtypeStruct((1, LAT), jnp.float32),
            jax.ShapeDtypeStruct((1, LAT), jnp.float32),
            jax.ShapeDtypeStruct((1, OUT), jnp.float32),
            jax.ShapeDtypeStruct((1, 1), jnp.float32),
        ],
    )(a, t, cn, mu_W, mu_b, lv_W, lv_b, pr_W1, pr_b1, pr_W2, pr_b2, eps)


def _sc2_body(adj_hbm, out_hbm, v_v, o_v):
    cid = lax.axis_index("c")
    sid = lax.axis_index("s")

    @pl.when(jnp.logical_and(cid == 0, sid == 0))
    def _():
        pltpu.sync_copy(adj_hbm, v_v)
        v0 = v_v[pl.ds(0, 16)]
        fill = jnp.full((16,), v0[0], jnp.float32)

        def _prefill(k, _):
            o_v[pl.ds(k * 16, 16)] = fill
            return 0
        lax.fori_loop(0, (C * C) // 16, _prefill, 0)

        def _pack(k, base):
            x = v_v[pl.ds(k * 16, 16)]
            m = x != 0.0
            pos = base + plsc.cumsum(m.astype(jnp.int32)) - 1
            plsc.store_scatter(o_v, [pos], x, mask=m)
            return base + plsc.all_reduce_population_count(m)
        lax.fori_loop(0, (C * C) // 16, _pack, jnp.zeros((16,), jnp.int32))
        pltpu.sync_copy(o_v, out_hbm)


_sc2 = functools.partial(
    pl.kernel,
    out_type=jax.ShapeDtypeStruct((C * C,), jnp.float32),
    mesh=plsc.VectorSubcoreMesh(core_axis_name="c", subcore_axis_name="s"),
    scratch_types=[
        pltpu.VMEM((C * C,), jnp.float32),
        pltpu.VMEM((C * C,), jnp.float32),
    ],
    compiler_params=pltpu.CompilerParams(needs_layout_passes=False, use_tc_tiling_on_sc=False),
)(_sc2_body)


def kernel(nodes, senders, receivers, edges, enc_W1, enc_b1, enc_W2, enc_b2,
           asn_W1, asn_b1, asn_W2, asn_b2, mu_W, mu_b, lv_W, lv_b,
           pr_W1, pr_b1, pr_W2, pr_b2):
    r2 = lambda b: b.reshape(1, -1)
    assignments, coarse_nodes = _tc1(
        nodes, enc_W1, r2(enc_b1), enc_W2, r2(enc_b2),
        asn_W1, r2(asn_b1), asn_W2, r2(asn_b2))

    pad = E_PAD - E
    zi = jnp.zeros((pad,), jnp.int32)
    send3 = jnp.concatenate([senders.astype(jnp.int32), zi]).reshape(NW, NCH, CH)
    recv3 = jnp.concatenate([receivers.astype(jnp.int32), zi]).reshape(NW, NCH, CH)
    ew3 = jnp.concatenate([edges.astype(jnp.float32).reshape(-1),
                           jnp.zeros((pad,), jnp.float32)]).reshape(NW, NCH, CH)
    t_partial = _sc1(send3, recv3, ew3, assignments)

    eps = jax.random.normal(jax.random.PRNGKey(0), (1, LAT))
    coarse_adj, mu, logvar, pred_y, nnz = _tc2(
        assignments, t_partial, coarse_nodes,
        mu_W, r2(mu_b), lv_W, r2(lv_b),
        pr_W1, r2(pr_b1), pr_W2, r2(pr_b2), eps)

    c_edges = lax.cond(
        nnz[0, 0] == float(C * C),
        lambda a: a.reshape(C * C, 1),
        lambda a: _sc2(a.reshape(C * C)).reshape(C * C, 1),
        coarse_adj)
    return (mu, logvar, pred_y, assignments, coarse_nodes, c_edges)
